# baseline (speedup 1.0000x reference)
import functools
import jax
import jax.numpy as jnp
from jax import lax
from jax.experimental import pallas as pl
from jax.experimental.pallas import tpu as pltpu
from jax.experimental.pallas import tpu_sc as plsc

N_FIELDS = 26
N_PAIRS = N_FIELDS * (N_FIELDS - 1) // 2
D = 128
L = 16
NV = D // L
B = 4096
NC = 2
NS = 16
NW = NC * NS
BPW = B // NW


def _compute_pairs(src, dst):
    off = 0
    for i in range(N_FIELDS - 1):
        a = [src[i, pl.ds(v * L, L)] for v in range(NV)]

        def jbody(j, c, a=a, i=i, off=off):
            row = off + j - (i + 1)
            for v in range(NV):
                dst[row, pl.ds(v * L, L)] = a[v] * src[j, pl.ds(v * L, L)]
            return c

        lax.fori_loop(i + 1, N_FIELDS, jbody, 0)
        off += N_FIELDS - 1 - i


def _sc_body(in_hbm, out_hbm, in_v, out_v, sem_in, sem_o0, sem_o1):
    wid = lax.axis_index("s") * NC + lax.axis_index("c")
    base = wid * BPW
    out_sems = (sem_o0, sem_o1)

    def step(t, c):
        for k in range(2):
            b = base + 2 * t + k
            pltpu.sync_copy(in_hbm.at[b], in_v)

            @pl.when(t > 0)
            def _wait(k=k):
                pltpu.make_async_copy(out_v.at[k], out_hbm.at[b], out_sems[k]).wait()

            _compute_pairs(in_v, out_v.at[k])
            pltpu.make_async_copy(out_v.at[k], out_hbm.at[b], out_sems[k]).start()
        return c

    lax.fori_loop(0, BPW // 2, step, 0)
    last = base + BPW - 1
    pltpu.make_async_copy(out_v.at[0], out_hbm.at[last - 1], sem_o0).wait()
    pltpu.make_async_copy(out_v.at[1], out_hbm.at[last], sem_o1).wait()


def kernel(inputs):
    return pl.kernel(
        _sc_body,
        out_type=jax.ShapeDtypeStruct((B, N_PAIRS, D), jnp.float32),
        mesh=plsc.VectorSubcoreMesh(core_axis_name="c", subcore_axis_name="s"),
        scratch_types=[
            pltpu.VMEM((N_FIELDS, D), jnp.float32),
            pltpu.VMEM((2, N_PAIRS, D), jnp.float32),
            pltpu.SemaphoreType.DMA,
            pltpu.SemaphoreType.DMA,
            pltpu.SemaphoreType.DMA,
        ],
    )(inputs)
